# Initial kernel scaffold; baseline (speedup 1.0000x reference)
import jax
import jax.numpy as jnp
from jax.experimental import pallas as pl


def kernel(embeddings, labels):
    raise NotImplementedError("write your pallas kernel here")


# speedup vs baseline: 83.2506x; 83.2506x over previous
import functools

import jax
import jax.numpy as jnp
from jax import lax
from jax.experimental import pallas as pl
from jax.experimental.pallas import tpu as pltpu
from jax.experimental.pallas import tpu_sc as plsc

_NUM_STEPS = 100
_MARGIN = 0.1
_TILE = 512
_NCODES = 256


def _tc_body(embr_ref, embc_ref, labr_ref, labc_ref, codes_ref, sums_ref):
    bi = pl.program_id(0)
    bj = pl.program_id(1)
    t = codes_ref.shape[0]

    er = embr_ref[...]
    ec = embc_ref[...]
    sr = 1.0 / jnp.maximum(jnp.sqrt(jnp.sum(er * er, axis=1, keepdims=True)), 1e-12)
    sc = 1.0 / jnp.maximum(jnp.sqrt(jnp.sum(ec * ec, axis=1, keepdims=True)), 1e-12)
    sim = lax.dot_general(er * sr, ec * sc, (((1,), (1,)), ((), ())),
                          preferred_element_type=jnp.float32)

    eq = labr_ref[...] == labc_ref[...]
    ri = lax.broadcasted_iota(jnp.int32, (t, t), 0) + bi * t
    cj = lax.broadcasted_iota(jnp.int32, (t, t), 1) + bj * t
    isdiag = ri == cj

    bin_idx = jnp.clip(jnp.floor((sim + 1.0) / 2.0 * _NUM_STEPS).astype(jnp.int32),
                       0, _NUM_STEPS - 1)
    code = jnp.where(eq, bin_idx + _NUM_STEPS, bin_idx)
    code = jnp.where(isdiag, _NCODES - 1, code)
    codes_ref[...] = code

    eqf = eq.astype(jnp.float32)
    s_all = jnp.sum(sim)
    s_eq = jnp.sum(jnp.where(eq, sim, 0.0))
    n_eq = jnp.sum(eqf)
    s_diag = jnp.sum(jnp.where(isdiag, sim, 0.0))

    row = lax.broadcasted_iota(jnp.int32, (8, 128), 0)
    lane = lax.broadcasted_iota(jnp.int32, (8, 128), 1)
    on_r0 = row == 0
    vec = (jnp.where(on_r0 & (lane == 0), s_all, 0.0)
           + jnp.where(on_r0 & (lane == 1), s_eq, 0.0)
           + jnp.where(on_r0 & (lane == 2), n_eq, 0.0)
           + jnp.where(on_r0 & (lane == 3), s_diag, 0.0))

    first = jnp.logical_and(bi == 0, bj == 0)

    @pl.when(first)
    def _():
        sums_ref[...] = vec

    @pl.when(jnp.logical_not(first))
    def _():
        sums_ref[...] += vec


def _tc_codes(emb, labels):
    b, _ = emb.shape
    nt = b // _TILE
    labr = labels.reshape(b, 1)
    labc = labels.reshape(1, b)
    return pl.pallas_call(
        _tc_body,
        grid=(nt, nt),
        in_specs=[
            pl.BlockSpec((_TILE, emb.shape[1]), lambda i, j: (i, 0)),
            pl.BlockSpec((_TILE, emb.shape[1]), lambda i, j: (j, 0)),
            pl.BlockSpec((_TILE, 1), lambda i, j: (i, 0)),
            pl.BlockSpec((1, _TILE), lambda i, j: (0, j)),
        ],
        out_specs=[
            pl.BlockSpec((_TILE, _TILE), lambda i, j: (i, j)),
            pl.BlockSpec((8, 128), lambda i, j: (0, 0)),
        ],
        out_shape=[
            jax.ShapeDtypeStruct((b, b), jnp.int32),
            jax.ShapeDtypeStruct((8, 128), jnp.float32),
        ],
    )(emb, emb, labr, labc)


def _make_sc_hist(total):
    info = plsc.get_sparse_core_info()
    nc, ns = info.num_cores, info.num_subcores
    nw = nc * ns
    words_per_w = total // nw
    ch = 32768
    nchunk = words_per_w // ch
    hist_words = 16 * _NCODES
    mesh = plsc.VectorSubcoreMesh(core_axis_name="c", subcore_axis_name="s")

    @functools.partial(
        pl.kernel, mesh=mesh,
        out_type=jax.ShapeDtypeStruct((nw * _NCODES,), jnp.float32),
        compiler_params=pltpu.CompilerParams(needs_layout_passes=False),
        scratch_types=[
            pltpu.VMEM((2, ch), jnp.int32),
            pltpu.VMEM((hist_words,), jnp.float32),
            pltpu.VMEM((_NCODES,), jnp.float32),
            pltpu.SemaphoreType.DMA,
            pltpu.SemaphoreType.DMA,
        ],
    )
    def sc_hist(codes_hbm, out_hbm, buf, hist, histred, sem0, sem1):
        wid = lax.axis_index("s") * nc + lax.axis_index("c")
        base = wid * words_per_w
        sems = (sem0, sem1)

        zero16 = jnp.zeros((16,), jnp.float32)

        def zinit(i, _):
            hist[pl.ds(i * 16, 16)] = zero16
            return 0

        lax.fori_loop(0, hist_words // 16, zinit, 0)

        ones16 = jnp.full((16,), 1.0, jnp.float32)
        laneoff = lax.iota(jnp.int32, 16) * _NCODES

        copies = []
        copies.append(pltpu.async_copy(
            codes_hbm.at[pl.ds(base, ch)], buf.at[0], sems[0]))

        for g in range(nchunk):
            bsel = g % 2
            if g + 1 < nchunk:
                copies.append(pltpu.async_copy(
                    codes_hbm.at[pl.ds(base + (g + 1) * ch, ch)],
                    buf.at[(g + 1) % 2], sems[(g + 1) % 2]))
            copies[g].wait()

            def body(k, _, bsel=bsel):
                kb = k * 128
                for u in range(8):
                    idx = buf[bsel, pl.ds(kb + u * 16, 16)]
                    plsc.addupdate_scatter(hist, [idx + laneoff], ones16)
                return 0

            lax.fori_loop(0, ch // 128, body, 0)

        for c in range(_NCODES // 16):
            acc = zero16
            for s in range(16):
                acc = acc + hist[pl.ds(s * _NCODES + c * 16, 16)]
            histred[pl.ds(c * 16, 16)] = acc

        pltpu.sync_copy(histred, out_hbm.at[pl.ds(wid * _NCODES, _NCODES)])

    return sc_hist, nw


def kernel(embeddings, labels):
    b = embeddings.shape[0]
    labels = labels.astype(jnp.int32)

    codes, sums = _tc_codes(embeddings, labels)

    sc_hist, nw = _make_sc_hist(b * b)
    partials = sc_hist(codes.reshape(-1))
    hist = jnp.sum(partials.reshape(nw, _NCODES), axis=0)

    neg_hist = hist[:_NUM_STEPS]
    pos_hist = hist[_NUM_STEPS:2 * _NUM_STEPS]
    pos_hist = pos_hist / (jnp.sum(pos_hist) + 1e-16)
    neg_hist = neg_hist / (jnp.sum(neg_hist) + 1e-16)
    overlap = jnp.sum(jnp.minimum(pos_hist, neg_hist))

    s_all = sums[0, 0]
    s_eq = sums[0, 1]
    n_eq = sums[0, 2]
    s_diag = sums[0, 3]
    bf = jnp.float32(b)
    pos_mean = (s_eq - s_diag) / (n_eq - bf)
    neg_mean = (s_all - s_eq) / (bf * bf - n_eq)

    return overlap + jax.nn.relu(_MARGIN - (pos_mean - neg_mean))


# speedup vs baseline: 135.0716x; 1.6225x over previous
import functools

import jax
import jax.numpy as jnp
from jax import lax
from jax.experimental import pallas as pl
from jax.experimental.pallas import tpu as pltpu
from jax.experimental.pallas import tpu_sc as plsc

_NUM_STEPS = 100
_MARGIN = 0.1
_TILE = 512
_NCODES = 256


def _tc_body(embr_ref, embc_ref, labr_ref, labc_ref, codes_ref, sums_ref):
    bi = pl.program_id(0)
    bj = pl.program_id(1)
    t = codes_ref.shape[0]

    er = embr_ref[...]
    ec = embc_ref[...]
    sr = 1.0 / jnp.maximum(jnp.sqrt(jnp.sum(er * er, axis=1, keepdims=True)), 1e-12)
    sc = 1.0 / jnp.maximum(jnp.sqrt(jnp.sum(ec * ec, axis=1, keepdims=True)), 1e-12)
    sim = lax.dot_general(er * sr, ec * sc, (((1,), (1,)), ((), ())),
                          preferred_element_type=jnp.float32)

    eq = labr_ref[...] == labc_ref[...]
    ri = lax.broadcasted_iota(jnp.int32, (t, t), 0) + bi * t
    cj = lax.broadcasted_iota(jnp.int32, (t, t), 1) + bj * t
    isdiag = ri == cj

    bin_idx = jnp.clip(jnp.floor((sim + 1.0) / 2.0 * _NUM_STEPS).astype(jnp.int32),
                       0, _NUM_STEPS - 1)
    code = jnp.where(eq, bin_idx + _NUM_STEPS, bin_idx)
    code = jnp.where(isdiag, _NCODES - 1, code)
    codes_ref[...] = code

    eqf = eq.astype(jnp.float32)
    s_all = jnp.sum(sim)
    s_eq = jnp.sum(jnp.where(eq, sim, 0.0))
    n_eq = jnp.sum(eqf)
    s_diag = jnp.sum(jnp.where(isdiag, sim, 0.0))

    row = lax.broadcasted_iota(jnp.int32, (8, 128), 0)
    lane = lax.broadcasted_iota(jnp.int32, (8, 128), 1)
    on_r0 = row == 0
    vec = (jnp.where(on_r0 & (lane == 0), s_all, 0.0)
           + jnp.where(on_r0 & (lane == 1), s_eq, 0.0)
           + jnp.where(on_r0 & (lane == 2), n_eq, 0.0)
           + jnp.where(on_r0 & (lane == 3), s_diag, 0.0))

    first = jnp.logical_and(bi == 0, bj == 0)

    @pl.when(first)
    def _():
        sums_ref[...] = vec

    @pl.when(jnp.logical_not(first))
    def _():
        sums_ref[...] += vec


def _tc_codes(emb, labels):
    b, _ = emb.shape
    nt = b // _TILE
    labr = labels.reshape(b, 1)
    labc = labels.reshape(1, b)
    return pl.pallas_call(
        _tc_body,
        grid=(nt, nt),
        in_specs=[
            pl.BlockSpec((_TILE, emb.shape[1]), lambda i, j: (i, 0)),
            pl.BlockSpec((_TILE, emb.shape[1]), lambda i, j: (j, 0)),
            pl.BlockSpec((_TILE, 1), lambda i, j: (i, 0)),
            pl.BlockSpec((1, _TILE), lambda i, j: (0, j)),
        ],
        out_specs=[
            pl.BlockSpec((_TILE, _TILE), lambda i, j: (i, j)),
            pl.BlockSpec((8, 128), lambda i, j: (0, 0)),
        ],
        out_shape=[
            jax.ShapeDtypeStruct((b, b), jnp.int32),
            jax.ShapeDtypeStruct((8, 128), jnp.float32),
        ],
    )(emb, emb, labr, labc)


def _make_sc_hist(total):
    info = plsc.get_sparse_core_info()
    nc, ns = info.num_cores, info.num_subcores
    nw = nc * ns
    words_per_w = total // nw
    ch = 32768
    nchunk = words_per_w // ch
    hist_words = 16 * _NCODES
    mesh = plsc.VectorSubcoreMesh(core_axis_name="c", subcore_axis_name="s")

    @functools.partial(
        pl.kernel, mesh=mesh,
        out_type=jax.ShapeDtypeStruct((nw * _NCODES,), jnp.float32),
        compiler_params=pltpu.CompilerParams(needs_layout_passes=False),
        scratch_types=[
            pltpu.VMEM((2, ch), jnp.int32),
            pltpu.VMEM((hist_words,), jnp.float32),
            pltpu.VMEM((_NCODES,), jnp.float32),
            pltpu.SemaphoreType.DMA,
            pltpu.SemaphoreType.DMA,
        ],
    )
    def sc_hist(codes_hbm, out_hbm, buf, hist, histred, sem0, sem1):
        wid = lax.axis_index("s") * nc + lax.axis_index("c")
        base = wid * words_per_w
        sems = (sem0, sem1)

        zero16 = jnp.zeros((16,), jnp.float32)

        def zinit(i, _):
            hist[pl.ds(i * 16, 16)] = zero16
            return 0

        lax.fori_loop(0, hist_words // 16, zinit, 0)

        ones16 = jnp.full((16,), 1.0, jnp.float32)
        laneoff = lax.iota(jnp.int32, 16) * _NCODES

        copies = []
        copies.append(pltpu.async_copy(
            codes_hbm.at[pl.ds(base, ch)], buf.at[0], sems[0]))

        for g in range(nchunk):
            bsel = g % 2
            if g + 1 < nchunk:
                copies.append(pltpu.async_copy(
                    codes_hbm.at[pl.ds(base + (g + 1) * ch, ch)],
                    buf.at[(g + 1) % 2], sems[(g + 1) % 2]))
            copies[g].wait()

            @plsc.parallel_loop(0, ch, step=128, unroll=4)
            def _(k, bsel=bsel):
                for u in range(8):
                    idx = buf[bsel, pl.ds(k + u * 16, 16)]
                    plsc.addupdate_scatter(hist, [idx + laneoff], ones16)

        for c in range(_NCODES // 16):
            acc = zero16
            for s in range(16):
                acc = acc + hist[pl.ds(s * _NCODES + c * 16, 16)]
            histred[pl.ds(c * 16, 16)] = acc

        pltpu.sync_copy(histred, out_hbm.at[pl.ds(wid * _NCODES, _NCODES)])

    return sc_hist, nw


def kernel(embeddings, labels):
    b = embeddings.shape[0]
    labels = labels.astype(jnp.int32)

    codes, sums = _tc_codes(embeddings, labels)

    sc_hist, nw = _make_sc_hist(b * b)
    partials = sc_hist(codes.reshape(-1))
    hist = jnp.sum(partials.reshape(nw, _NCODES), axis=0)

    neg_hist = hist[:_NUM_STEPS]
    pos_hist = hist[_NUM_STEPS:2 * _NUM_STEPS]
    pos_hist = pos_hist / (jnp.sum(pos_hist) + 1e-16)
    neg_hist = neg_hist / (jnp.sum(neg_hist) + 1e-16)
    overlap = jnp.sum(jnp.minimum(pos_hist, neg_hist))

    s_all = sums[0, 0]
    s_eq = sums[0, 1]
    n_eq = sums[0, 2]
    s_diag = sums[0, 3]
    bf = jnp.float32(b)
    pos_mean = (s_eq - s_diag) / (n_eq - bf)
    neg_mean = (s_all - s_eq) / (bf * bf - n_eq)

    return overlap + jax.nn.relu(_MARGIN - (pos_mean - neg_mean))
